# baseline (speedup 1.0000x reference)
import functools

import jax
import jax.numpy as jnp
from jax import lax
from jax.experimental import pallas as pl
from jax.experimental.pallas import tpu as pltpu
from jax.experimental.pallas import tpu_sc as plsc

_B = 16384
_D = 32
_NU = 26
_NW = 26
_BLK = 2048


def _sc_gather3(idx_pat, idx_meas, idx_t, pat_lat, meas_lat, time_lat):
    info = plsc.get_sparse_core_info()
    nc, ns = info.num_cores, info.num_subcores
    nw = nc * ns
    bpw = _B // nw

    mesh = plsc.VectorSubcoreMesh(core_axis_name="c", subcore_axis_name="s")

    @functools.partial(
        pl.kernel,
        mesh=mesh,
        compiler_params=pltpu.CompilerParams(use_tc_tiling_on_sc=False),
        out_type=(
            jax.ShapeDtypeStruct((_B, _D), jnp.float32),
            jax.ShapeDtypeStruct((_B, _D), jnp.float32),
            jax.ShapeDtypeStruct((_B, _D), jnp.float32),
        ),
        scratch_types=[
            pltpu.VMEM((bpw,), jnp.int32),
            pltpu.VMEM((bpw,), jnp.int32),
            pltpu.VMEM((bpw,), jnp.int32),
            pltpu.VMEM((bpw, _D), jnp.float32),
            pltpu.VMEM((bpw, _D), jnp.float32),
            pltpu.VMEM((bpw, _D), jnp.float32),
            pltpu.SemaphoreType.DMA,
            pltpu.SemaphoreType.DMA,
            pltpu.SemaphoreType.DMA,
        ],
    )
    def gather_kernel(ip_hbm, im_hbm, it_hbm, pt_hbm, mt_hbm, tt_hbm,
                      op_hbm, om_hbm, ot_hbm,
                      ip_v, im_v, it_v, p_v, m_v, t_v, sp, sm, st):
        wid = lax.axis_index("s") * nc + lax.axis_index("c")
        base = wid * bpw
        pltpu.sync_copy(ip_hbm.at[pl.ds(base, bpw)], ip_v)
        pltpu.sync_copy(im_hbm.at[pl.ds(base, bpw)], im_v)
        pltpu.sync_copy(it_hbm.at[pl.ds(base, bpw)], it_v)
        cp = pltpu.async_copy(pt_hbm.at[ip_v], p_v, sp)
        cm = pltpu.async_copy(mt_hbm.at[im_v], m_v, sm)
        ct = pltpu.async_copy(tt_hbm.at[it_v], t_v, st)
        cp.wait()
        cm.wait()
        ct.wait()
        pltpu.sync_copy(p_v, op_hbm.at[pl.ds(base, bpw)])
        pltpu.sync_copy(m_v, om_hbm.at[pl.ds(base, bpw)])
        pltpu.sync_copy(t_v, ot_hbm.at[pl.ds(base, bpw)])

    return gather_kernel(idx_pat, idx_meas, idx_t, pat_lat, meas_lat, time_lat)


def _tc_combine(pat, meas, tim, cov_u, cov_w, beta_u, beta_w):
    nblk = _B // _BLK

    def body(pat_ref, meas_ref, tim_ref, cu_ref, cw_ref, bu_ref, bw_ref, out_ref):
        u = jnp.dot(cu_ref[...], bu_ref[...], preferred_element_type=jnp.float32)
        w = jnp.dot(cw_ref[...], bw_ref[...], preferred_element_type=jnp.float32)
        prod = (pat_ref[...] + u) * meas_ref[...] * (tim_ref[...] + w)
        out_ref[...] = jnp.sum(prod, axis=1).reshape(1, 1, _BLK)

    out = pl.pallas_call(
        body,
        grid=(nblk,),
        in_specs=[
            pl.BlockSpec((_BLK, _D), lambda i: (i, 0)),
            pl.BlockSpec((_BLK, _D), lambda i: (i, 0)),
            pl.BlockSpec((_BLK, _D), lambda i: (i, 0)),
            pl.BlockSpec((_BLK, _NU), lambda i: (i, 0)),
            pl.BlockSpec((_BLK, _NW), lambda i: (i, 0)),
            pl.BlockSpec((_NU, _D), lambda i: (0, 0)),
            pl.BlockSpec((_NW, _D), lambda i: (0, 0)),
        ],
        out_specs=pl.BlockSpec((1, 1, _BLK), lambda i: (i, 0, 0)),
        out_shape=jax.ShapeDtypeStruct((nblk, 1, _BLK), jnp.float32),
    )(pat, meas, tim, cov_u, cov_w, beta_u, beta_w)
    return out.reshape(_B)


def kernel(idx_pat, idx_meas, idx_t, cov_u, cov_w, pat_lat, meas_lat, time_lat,
           beta_u, beta_w):
    ip = idx_pat.astype(jnp.int32)
    im = idx_meas.astype(jnp.int32)
    it = idx_t.astype(jnp.int32)
    pat, meas, tim = _sc_gather3(ip, im, it, pat_lat, meas_lat, time_lat)
    return _tc_combine(pat, meas, tim, cov_u, cov_w, beta_u, beta_w)
